# baseline (speedup 1.0000x reference)
import functools

import jax
import jax.numpy as jnp
from jax import lax
from jax.experimental import pallas as pl
from jax.experimental.pallas import tpu as pltpu
from jax.experimental.pallas import tpu_sc as plsc

H = W = 256
S = 48
START = 3
NSEG = 35
NDEC = NSEG - 1
NC = 2
NSUB = 16
NWORK = NC * NSUB
L = 16


def _make_mesh():
    return plsc.VectorSubcoreMesh(core_axis_name="c", subcore_axis_name="s")


def _wid():
    return lax.axis_index("s") * NC + lax.axis_index("c")


def _a_body(B, N, seg_len, ch, ev, alongx, alongy, packed, ebuf, pbuf, hx, hy):
    wid = _wid()
    units = (B * S) // NWORK
    lane = lax.iota(jnp.int32, L)
    lane5 = lane * 5
    lane256 = lane * 256
    ones = jnp.ones((L,), jnp.int32)
    zeros16 = jnp.zeros((L,), jnp.int32)
    n_chunks = seg_len // ch

    for uu in range(units):
        u = wid * units + uu
        b = u // S
        seg = u % S
        pbase = b * N + seg * seg_len

        def zb(i, _):
            hx[pl.ds(i * L, L)] = zeros16
            hy[pl.ds(i * L, L)] = zeros16
            return _
        lax.fori_loop(0, 256, zb, None)

        def chunk_body(c, _):
            off = seg * seg_len + c * ch
            pltpu.sync_copy(ev.at[b, pl.ds(off, ch)], ebuf)

            def inner(i, _i):
                r = i * L + lane
                xf = plsc.load_gather(ebuf, [r, zeros16])
                yf = plsc.load_gather(ebuf, [r, ones])
                xi = xf.astype(jnp.int32)
                yi = yf.astype(jnp.int32)
                plsc.addupdate_scatter(hx, [lane256 + xi], ones)
                plsc.addupdate_scatter(hy, [lane256 + yi], ones)
                pbuf[pl.ds(i * L, L)] = xi + yi * 256
                return _i
            lax.fori_loop(0, ch // L, inner, None)
            pltpu.sync_copy(pbuf, packed.at[pl.ds(pbase + c * ch, ch)])
            return _
        lax.fori_loop(0, n_chunks, chunk_body, None)

        def red(g, _):
            accx = hx[pl.ds(g * L, L)]
            accy = hy[pl.ds(g * L, L)]
            for l in range(1, L):
                accx = accx + hx[pl.ds(l * 256 + g * L, L)]
                accy = accy + hy[pl.ds(l * 256 + g * L, L)]
            hx[pl.ds(g * L, L)] = accx
            hy[pl.ds(g * L, L)] = accy
            return _
        lax.fori_loop(0, 16, red, None)
        pltpu.sync_copy(hx.at[pl.ds(0, 256)], alongx.at[pl.ds(u * 256, 256)])
        pltpu.sync_copy(hy.at[pl.ds(0, 256)], alongy.at[pl.ds(u * 256, 256)])


def _median10(v):
    le = jnp.sum((v[..., None, :] <= v[..., :, None]).astype(jnp.int32), axis=-1)
    big = jnp.full_like(v, 3.4e38)
    os5 = jnp.min(jnp.where(le >= 5, v, big), axis=-1)
    os6 = jnp.min(jnp.where(le >= 6, v, big), axis=-1)
    return (os5 + os6) / 2


def _outlier_first(v, thresh):
    med = _median10(v)
    diff = jnp.abs(v - med[..., None])
    mad = _median10(diff)
    mad = jnp.where(mad == 0, jnp.float32(1e-12), mad)
    mz = jnp.float32(0.6745) * diff[..., 0] / mad
    return mz > thresh


def _b_body(seg_len, ax_ref, ay_ref, axb_ref, ayb_ref, skip_ref):
    B = ax_ref.shape[0]
    D = 256

    def stats(cnt):
        a = cnt.astype(jnp.float32)
        n = S * D
        mean = jnp.sum(a, axis=(1, 2)) / jnp.float32(n)
        var = jnp.sum((a - mean[:, None, None]) ** 2, axis=(1, 2)) / jnp.float32(n - 1)
        cvv = mean + 3.0 * jnp.sqrt(var)
        a = jnp.clip(a, 0.0, cvv[:, None, None])
        zc = jnp.zeros((B, S, 2), jnp.float32)
        a2 = jnp.concatenate([zc, a, zc], axis=2)
        zr = jnp.zeros((B, 2, D + 4), jnp.float32)
        a3 = jnp.concatenate([zr, a2, zr], axis=1)
        acc = jnp.zeros_like(a)
        for di in range(5):
            for dj in range(5):
                acc = acc + a3[:, di:di + S, dj:dj + D]
        blur = acc * jnp.float32(0.04)
        w = lax.broadcasted_iota(jnp.int32, (1, 1, D), 2).astype(jnp.float32)
        mean_a = jnp.sum(blur * w, axis=-1) / jnp.float32(seg_len)
        start = mean_a[:, START:START + 1]
        aligned = jnp.round((mean_a - start) - (jnp.float32(D // 2) - start))
        return mean_a, aligned.astype(jnp.int32)

    mx, alx = stats(ax_ref[...])
    my, aly = stats(ay_ref[...])
    axb_ref[...] = jnp.broadcast_to(alx[:, :, None], (B, S, L))
    ayb_ref[...] = jnp.broadcast_to(aly[:, :, None], (B, S, L))
    wx = jnp.concatenate([mx[:, None, si:si + 10] for si in range(4, 4 + NDEC)], axis=1)
    wy = jnp.concatenate([my[:, None, si:si + 10] for si in range(4, 4 + NDEC)], axis=1)
    skip = jnp.logical_or(_outlier_first(wx, 2.0), _outlier_first(wy, 2.0))
    skip_ref[...] = skip.astype(jnp.int32)


def _c_body(B, N, seg_len, ch, packed, axb, ayb, occ, pbuf, abx, aby, occ_s):
    wid = _wid()
    ones = jnp.ones((L,), jnp.int32)
    zeros16 = jnp.zeros((L,), jnp.int32)
    n_units = B * NSEG
    n_rounds = (n_units + NWORK - 1) // NWORK
    n_chunks = seg_len // ch

    for rnd in range(n_rounds):
        u = wid + rnd * NWORK

        @pl.when(u < n_units)
        def _():
            b = u // NSEG
            kk = u % NSEG
            si = kk + START
            pltpu.sync_copy(axb.at[pl.ds((b * S + si) * L, L)], abx)
            pltpu.sync_copy(ayb.at[pl.ds((b * S + si) * L, L)], aby)
            axv = abx[...]
            ayv = aby[...]

            def zb(i, _):
                occ_s[pl.ds(i * L, L)] = zeros16
                return _
            lax.fori_loop(0, 16384 // L, zb, None)

            def chunk_body(c, _):
                pltpu.sync_copy(
                    packed.at[pl.ds(b * N + si * seg_len + c * ch, ch)], pbuf)

                def inner(i, _i):
                    p = pbuf[pl.ds(i * L, L)]
                    xi = lax.bitwise_and(p, 255)
                    yi = lax.shift_right_logical(p, 8)
                    xs = jnp.minimum(jnp.maximum(xi - axv, 0), 255)
                    ys = jnp.minimum(jnp.maximum(yi - ayv, 0), 255)
                    cv = (lax.shift_right_logical(ys, 1) * 128
                          + lax.shift_right_logical(xs, 1))
                    plsc.store_scatter(occ_s, [cv], ones)
                    return _i
                lax.fori_loop(0, ch // L, inner, None)
                return _
            lax.fori_loop(0, n_chunks, chunk_body, None)
            pltpu.sync_copy(occ_s, occ.at[pl.ds(u * 16384, 16384)])


def _d_body(occ_ref, skip_ref, wb_ref):
    B = occ_ref.shape[0]
    skip = skip_ref[...] != 0
    ver = occ_ref[:, 0, :, :]
    s_old = jnp.sum(ver, axis=(1, 2)).astype(jnp.float32)
    active = jnp.ones((B,), jnp.bool_)
    ws = [jnp.ones((B,), jnp.int32)]
    for k in range(1, NSEG):
        o = occ_ref[:, k, :, :]
        cand = jnp.maximum(ver, o)
        s = jnp.sum(cand, axis=(1, 2)).astype(jnp.float32)
        ni = s - s_old
        keep = (ni / s) >= jnp.float32(0.1)
        sk = skip[:, k - 1]
        ap = jnp.logical_and(active, jnp.logical_and(jnp.logical_not(sk), keep))
        ver = jnp.where(ap[:, None, None], cand, ver)
        s_old = jnp.where(ap, s, s_old)
        active = jnp.logical_and(active, jnp.logical_or(sk, keep))
        ws.append(ap.astype(jnp.int32))
    wmat = jnp.stack(ws, axis=1)
    wb_ref[...] = jnp.broadcast_to(wmat[:, :, None], (B, NSEG, L))


def _e_body(B, N, seg_len, ch, tiles_per_b, packed, axb, ayb, wb, parts,
            pbuf, abx, aby, wbuf, cont_s):
    segs_per_tile = -(-NSEG // tiles_per_b)
    wid = _wid()
    ones = jnp.ones((L,), jnp.int32)
    zeros16 = jnp.zeros((L,), jnp.int32)
    n_chunks = seg_len // ch

    @pl.when(wid < B * tiles_per_b)
    def _():
        b = wid // tiles_per_b
        r = wid % tiles_per_b

        def zb(i, _):
            cont_s[pl.ds(i * L, L)] = zeros16
            return _
        lax.fori_loop(0, 65536 // L, zb, None)

        for j in range(segs_per_tile):
            kk_s = None

            kk = r * segs_per_tile + j
            si = kk + START

            @pl.when(kk < NSEG)
            def _():
                pltpu.sync_copy(wb.at[pl.ds((b * NSEG + kk) * L, L)], wbuf)
                wsum = jnp.sum(wbuf[...])

                @pl.when(wsum > 0)
                def _():
                    pltpu.sync_copy(axb.at[pl.ds((b * S + si) * L, L)], abx)
                    pltpu.sync_copy(ayb.at[pl.ds((b * S + si) * L, L)], aby)
                    axv = abx[...]
                    ayv = aby[...]

                    def chunk_body(c, _):
                        pltpu.sync_copy(
                            packed.at[pl.ds(b * N + si * seg_len + c * ch, ch)],
                            pbuf)

                        def inner(i, _i):
                            p = pbuf[pl.ds(i * L, L)]
                            xi = lax.bitwise_and(p, 255)
                            yi = lax.shift_right_logical(p, 8)
                            xs = jnp.minimum(jnp.maximum(xi - axv, 0), 255)
                            ys = jnp.minimum(jnp.maximum(yi - ayv, 0), 255)
                            plsc.addupdate_scatter(cont_s, [ys * 256 + xs], ones)
                            return _i
                        lax.fori_loop(0, ch // L, inner, None)
                        return _
                    lax.fori_loop(0, n_chunks, chunk_body, None)
        pltpu.sync_copy(cont_s, parts.at[pl.ds(wid * 65536, 65536)])


def _f_body(parts_ref, out_ref):
    B = parts_ref.shape[0]
    cont = jnp.sum(parts_ref[...], axis=1).astype(jnp.float32)
    n = H * W
    mean = jnp.sum(cont, axis=(1, 2)) / jnp.float32(n)
    var = jnp.sum((cont - mean[:, None, None]) ** 2, axis=(1, 2)) / jnp.float32(n - 1)
    cv = mean + 3.0 * jnp.sqrt(var)
    c = jnp.clip(cont, 0.0, cv[:, None, None]) / cv[:, None, None]
    out_ref[...] = c[:, None, :, :]


@functools.lru_cache(maxsize=4)
def _build(B, N):
    seg_len = N // S
    ch_a = 800 if seg_len % 800 == 0 else seg_len
    ch_ce = 4000 if seg_len % 4000 == 0 else seg_len
    tiles_per_b = NWORK // B
    mesh = _make_mesh()

    phase_a = pl.kernel(
        functools.partial(_a_body, B, N, seg_len, ch_a),
        out_type=[jax.ShapeDtypeStruct((B * S * 256,), jnp.int32),
                  jax.ShapeDtypeStruct((B * S * 256,), jnp.int32),
                  jax.ShapeDtypeStruct((B * N,), jnp.int32)],
        mesh=mesh,
        compiler_params=pltpu.CompilerParams(needs_layout_passes=False),
        scratch_types=[pltpu.VMEM((ch_a, 5), jnp.float32),
                       pltpu.VMEM((ch_a,), jnp.int32),
                       pltpu.VMEM((16 * 256,), jnp.int32),
                       pltpu.VMEM((16 * 256,), jnp.int32)],
    )

    phase_b = pl.pallas_call(
        functools.partial(_b_body, seg_len),
        out_shape=[jax.ShapeDtypeStruct((B, S, L), jnp.int32),
                   jax.ShapeDtypeStruct((B, S, L), jnp.int32),
                   jax.ShapeDtypeStruct((B, NDEC), jnp.int32)],
    )

    phase_c = pl.kernel(
        functools.partial(_c_body, B, N, seg_len, ch_ce),
        out_type=[jax.ShapeDtypeStruct((B * NSEG * 16384,), jnp.int32)],
        mesh=mesh,
        compiler_params=pltpu.CompilerParams(needs_layout_passes=False),
        scratch_types=[pltpu.VMEM((ch_ce,), jnp.int32),
                       pltpu.VMEM((L,), jnp.int32),
                       pltpu.VMEM((L,), jnp.int32),
                       pltpu.VMEM((16384,), jnp.int32)],
    )

    phase_d = pl.pallas_call(
        _d_body,
        out_shape=[jax.ShapeDtypeStruct((B, NSEG, L), jnp.int32)],
    )

    phase_e = pl.kernel(
        functools.partial(_e_body, B, N, seg_len, ch_ce, tiles_per_b),
        out_type=[jax.ShapeDtypeStruct((B * tiles_per_b * 65536,), jnp.int32)],
        mesh=mesh,
        compiler_params=pltpu.CompilerParams(needs_layout_passes=False),
        scratch_types=[pltpu.VMEM((ch_ce,), jnp.int32),
                       pltpu.VMEM((L,), jnp.int32),
                       pltpu.VMEM((L,), jnp.int32),
                       pltpu.VMEM((L,), jnp.int32),
                       pltpu.VMEM((65536,), jnp.int32)],
    )

    phase_f = pl.pallas_call(
        _f_body,
        out_shape=jax.ShapeDtypeStruct((B, 1, H, W), jnp.float32),
    )

    return phase_a, phase_b, phase_c, phase_d, phase_e, phase_f


def kernel(events):
    B, N, _ = events.shape
    tiles_per_b = NWORK // B
    phase_a, phase_b, phase_c, phase_d, phase_e, phase_f = _build(B, N)
    alongx, alongy, packed = phase_a(events)
    axb, ayb, skip = phase_b(alongx.reshape(B, S, 256), alongy.reshape(B, S, 256))
    (occ,) = phase_c(packed, axb.reshape(-1), ayb.reshape(-1))
    (wb,) = phase_d(occ.reshape(B, NSEG, 128, 128), skip)
    (parts,) = phase_e(packed, axb.reshape(-1), ayb.reshape(-1), wb.reshape(-1))
    return phase_f(parts.reshape(B, tiles_per_b, 256, 256))


# speedup vs baseline: 5.1310x; 5.1310x over previous
import functools

import jax
import jax.numpy as jnp
from jax import lax
from jax.experimental import pallas as pl
from jax.experimental.pallas import tpu as pltpu
from jax.experimental.pallas import tpu_sc as plsc

H = W = 256
S = 48
START = 3
NSEG = 35
NDEC = NSEG - 1
NC = 2
NSUB = 16
NWORK = NC * NSUB
L = 16


def _make_mesh():
    return plsc.VectorSubcoreMesh(core_axis_name="c", subcore_axis_name="s")


def _wid():
    return lax.axis_index("s") * NC + lax.axis_index("c")


def _a_body(B, N, seg_len, ch, xp, yp, alongx, alongy, packed, xbuf, ybuf,
            pbuf, hx, hy):
    wid = _wid()
    units = (B * S) // NWORK
    lane = lax.iota(jnp.int32, L)
    lane256 = lane * 256
    ones = jnp.ones((L,), jnp.int32)
    zeros16 = jnp.zeros((L,), jnp.int32)
    n_chunks = seg_len // ch

    for uu in range(units):
        u = wid * units + uu
        base = u * seg_len

        def zb(i, _):
            hx[pl.ds(i * L, L)] = zeros16
            hy[pl.ds(i * L, L)] = zeros16
            return _
        lax.fori_loop(0, 256, zb, None)

        def chunk_body(c, _):
            off = base + c * ch
            pltpu.sync_copy(xp.at[pl.ds(off, ch)], xbuf)
            pltpu.sync_copy(yp.at[pl.ds(off, ch)], ybuf)

            def inner(i, _i):
                xi = xbuf[pl.ds(i * L, L)].astype(jnp.int32)
                yi = ybuf[pl.ds(i * L, L)].astype(jnp.int32)
                plsc.addupdate_scatter(hx, [lane256 + xi], ones)
                plsc.addupdate_scatter(hy, [lane256 + yi], ones)
                pbuf[pl.ds(i * L, L)] = xi + yi * 256
                return _i
            lax.fori_loop(0, ch // L, inner, None)
            pltpu.sync_copy(pbuf, packed.at[pl.ds(off, ch)])
            return _
        lax.fori_loop(0, n_chunks, chunk_body, None)

        def red(g, _):
            accx = hx[pl.ds(g * L, L)]
            accy = hy[pl.ds(g * L, L)]
            for l in range(1, L):
                accx = accx + hx[pl.ds(l * 256 + g * L, L)]
                accy = accy + hy[pl.ds(l * 256 + g * L, L)]
            hx[pl.ds(g * L, L)] = accx
            hy[pl.ds(g * L, L)] = accy
            return _
        lax.fori_loop(0, 16, red, None)
        pltpu.sync_copy(hx.at[pl.ds(0, 256)], alongx.at[pl.ds(u * 256, 256)])
        pltpu.sync_copy(hy.at[pl.ds(0, 256)], alongy.at[pl.ds(u * 256, 256)])


def _median10(v):
    le = jnp.sum((v[..., None, :] <= v[..., :, None]).astype(jnp.int32), axis=-1)
    big = jnp.full_like(v, 3.4e38)
    os5 = jnp.min(jnp.where(le >= 5, v, big), axis=-1)
    os6 = jnp.min(jnp.where(le >= 6, v, big), axis=-1)
    return (os5 + os6) / 2


def _outlier_first(v, thresh):
    med = _median10(v)
    diff = jnp.abs(v - med[..., None])
    mad = _median10(diff)
    mad = jnp.where(mad == 0, jnp.float32(1e-12), mad)
    mz = jnp.float32(0.6745) * diff[..., 0] / mad
    return mz > thresh


def _b_body(seg_len, ax_ref, ay_ref, axb_ref, ayb_ref, skip_ref):
    B = ax_ref.shape[0]
    D = 256

    def stats(cnt):
        a = cnt.astype(jnp.float32)
        n = S * D
        mean = jnp.sum(a, axis=(1, 2)) / jnp.float32(n)
        var = jnp.sum((a - mean[:, None, None]) ** 2, axis=(1, 2)) / jnp.float32(n - 1)
        cvv = mean + 3.0 * jnp.sqrt(var)
        a = jnp.clip(a, 0.0, cvv[:, None, None])
        zc = jnp.zeros((B, S, 2), jnp.float32)
        a2 = jnp.concatenate([zc, a, zc], axis=2)
        zr = jnp.zeros((B, 2, D + 4), jnp.float32)
        a3 = jnp.concatenate([zr, a2, zr], axis=1)
        acc = jnp.zeros_like(a)
        for di in range(5):
            for dj in range(5):
                acc = acc + a3[:, di:di + S, dj:dj + D]
        blur = acc * jnp.float32(0.04)
        w = lax.broadcasted_iota(jnp.int32, (1, 1, D), 2).astype(jnp.float32)
        mean_a = jnp.sum(blur * w, axis=-1) / jnp.float32(seg_len)
        start = mean_a[:, START:START + 1]
        aligned = jnp.round((mean_a - start) - (jnp.float32(D // 2) - start))
        return mean_a, aligned.astype(jnp.int32)

    mx, alx = stats(ax_ref[...])
    my, aly = stats(ay_ref[...])
    axb_ref[...] = jnp.broadcast_to(alx[:, :, None], (B, S, L))
    ayb_ref[...] = jnp.broadcast_to(aly[:, :, None], (B, S, L))
    wx = jnp.concatenate([mx[:, None, si:si + 10] for si in range(4, 4 + NDEC)], axis=1)
    wy = jnp.concatenate([my[:, None, si:si + 10] for si in range(4, 4 + NDEC)], axis=1)
    skip = jnp.logical_or(_outlier_first(wx, 2.0), _outlier_first(wy, 2.0))
    skip_ref[...] = skip.astype(jnp.int32)


def _c_body(B, N, seg_len, ch, packed, axb, ayb, occ, pbuf, abx, aby, occ_s):
    wid = _wid()
    ones = jnp.ones((L,), jnp.int32)
    zeros16 = jnp.zeros((L,), jnp.int32)
    n_units = B * NSEG
    n_rounds = (n_units + NWORK - 1) // NWORK
    n_chunks = seg_len // ch

    for rnd in range(n_rounds):
        u = wid + rnd * NWORK

        @pl.when(u < n_units)
        def _():
            b = u // NSEG
            kk = u % NSEG
            si = kk + START
            pltpu.sync_copy(axb.at[pl.ds((b * S + si) * L, L)], abx)
            pltpu.sync_copy(ayb.at[pl.ds((b * S + si) * L, L)], aby)
            axv = abx[...]
            ayv = aby[...]

            def zb(i, _):
                occ_s[pl.ds(i * L, L)] = zeros16
                return _
            lax.fori_loop(0, 16384 // L, zb, None)

            def chunk_body(c, _):
                pltpu.sync_copy(
                    packed.at[pl.ds(b * N + si * seg_len + c * ch, ch)], pbuf)

                def inner(i, _i):
                    p = pbuf[pl.ds(i * L, L)]
                    xi = lax.bitwise_and(p, 255)
                    yi = lax.shift_right_logical(p, 8)
                    xs = jnp.minimum(jnp.maximum(xi - axv, 0), 255)
                    ys = jnp.minimum(jnp.maximum(yi - ayv, 0), 255)
                    cv = (lax.shift_right_logical(ys, 1) * 128
                          + lax.shift_right_logical(xs, 1))
                    plsc.store_scatter(occ_s, [cv], ones)
                    return _i
                lax.fori_loop(0, ch // L, inner, None)
                return _
            lax.fori_loop(0, n_chunks, chunk_body, None)
            pltpu.sync_copy(occ_s, occ.at[pl.ds(u * 16384, 16384)])


def _d_body(occ_ref, skip_ref, wb_ref):
    B = occ_ref.shape[0]
    skip = skip_ref[...] != 0
    ver = occ_ref[:, 0, :, :]
    s_old = jnp.sum(ver, axis=(1, 2)).astype(jnp.float32)
    active = jnp.ones((B,), jnp.bool_)
    ws = [jnp.ones((B,), jnp.int32)]
    for k in range(1, NSEG):
        o = occ_ref[:, k, :, :]
        cand = jnp.maximum(ver, o)
        s = jnp.sum(cand, axis=(1, 2)).astype(jnp.float32)
        ni = s - s_old
        keep = (ni / s) >= jnp.float32(0.1)
        sk = skip[:, k - 1]
        ap = jnp.logical_and(active, jnp.logical_and(jnp.logical_not(sk), keep))
        ver = jnp.where(ap[:, None, None], cand, ver)
        s_old = jnp.where(ap, s, s_old)
        active = jnp.logical_and(active, jnp.logical_or(sk, keep))
        ws.append(ap.astype(jnp.int32))
    wmat = jnp.stack(ws, axis=1)
    wb_ref[...] = jnp.broadcast_to(wmat[:, :, None], (B, NSEG, L))


def _e_body(B, N, seg_len, ch, tiles_per_b, packed, axb, ayb, wb, parts,
            pbuf, abx, aby, wbuf, cont_s):
    segs_per_tile = -(-NSEG // tiles_per_b)
    wid = _wid()
    ones = jnp.ones((L,), jnp.int32)
    zeros16 = jnp.zeros((L,), jnp.int32)
    n_chunks = seg_len // ch

    @pl.when(wid < B * tiles_per_b)
    def _():
        b = wid // tiles_per_b
        r = wid % tiles_per_b

        def zb(i, _):
            cont_s[pl.ds(i * L, L)] = zeros16
            return _
        lax.fori_loop(0, 65536 // L, zb, None)

        for j in range(segs_per_tile):
            kk_s = None

            kk = r * segs_per_tile + j
            si = kk + START

            @pl.when(kk < NSEG)
            def _():
                pltpu.sync_copy(wb.at[pl.ds((b * NSEG + kk) * L, L)], wbuf)
                wsum = jnp.sum(wbuf[...])

                @pl.when(wsum > 0)
                def _():
                    pltpu.sync_copy(axb.at[pl.ds((b * S + si) * L, L)], abx)
                    pltpu.sync_copy(ayb.at[pl.ds((b * S + si) * L, L)], aby)
                    axv = abx[...]
                    ayv = aby[...]

                    def chunk_body(c, _):
                        pltpu.sync_copy(
                            packed.at[pl.ds(b * N + si * seg_len + c * ch, ch)],
                            pbuf)

                        def inner(i, _i):
                            p = pbuf[pl.ds(i * L, L)]
                            xi = lax.bitwise_and(p, 255)
                            yi = lax.shift_right_logical(p, 8)
                            xs = jnp.minimum(jnp.maximum(xi - axv, 0), 255)
                            ys = jnp.minimum(jnp.maximum(yi - ayv, 0), 255)
                            plsc.addupdate_scatter(cont_s, [ys * 256 + xs], ones)
                            return _i
                        lax.fori_loop(0, ch // L, inner, None)
                        return _
                    lax.fori_loop(0, n_chunks, chunk_body, None)
        pltpu.sync_copy(cont_s, parts.at[pl.ds(wid * 65536, 65536)])


def _f_body(parts_ref, out_ref):
    B = parts_ref.shape[0]
    cont = jnp.sum(parts_ref[...], axis=1).astype(jnp.float32)
    n = H * W
    mean = jnp.sum(cont, axis=(1, 2)) / jnp.float32(n)
    var = jnp.sum((cont - mean[:, None, None]) ** 2, axis=(1, 2)) / jnp.float32(n - 1)
    cv = mean + 3.0 * jnp.sqrt(var)
    c = jnp.clip(cont, 0.0, cv[:, None, None]) / cv[:, None, None]
    out_ref[...] = c[:, None, :, :]


@functools.lru_cache(maxsize=4)
def _build(B, N):
    seg_len = N // S
    ch_a = 2000 if seg_len % 2000 == 0 else seg_len
    ch_ce = 4000 if seg_len % 4000 == 0 else seg_len
    tiles_per_b = NWORK // B
    mesh = _make_mesh()

    phase_a = pl.kernel(
        functools.partial(_a_body, B, N, seg_len, ch_a),
        out_type=[jax.ShapeDtypeStruct((B * S * 256,), jnp.int32),
                  jax.ShapeDtypeStruct((B * S * 256,), jnp.int32),
                  jax.ShapeDtypeStruct((B * N,), jnp.int32)],
        mesh=mesh,
        compiler_params=pltpu.CompilerParams(needs_layout_passes=False),
        scratch_types=[pltpu.VMEM((ch_a,), jnp.float32),
                       pltpu.VMEM((ch_a,), jnp.float32),
                       pltpu.VMEM((ch_a,), jnp.int32),
                       pltpu.VMEM((16 * 256,), jnp.int32),
                       pltpu.VMEM((16 * 256,), jnp.int32)],
    )

    phase_b = pl.pallas_call(
        functools.partial(_b_body, seg_len),
        out_shape=[jax.ShapeDtypeStruct((B, S, L), jnp.int32),
                   jax.ShapeDtypeStruct((B, S, L), jnp.int32),
                   jax.ShapeDtypeStruct((B, NDEC), jnp.int32)],
    )

    phase_c = pl.kernel(
        functools.partial(_c_body, B, N, seg_len, ch_ce),
        out_type=[jax.ShapeDtypeStruct((B * NSEG * 16384,), jnp.int32)],
        mesh=mesh,
        compiler_params=pltpu.CompilerParams(needs_layout_passes=False),
        scratch_types=[pltpu.VMEM((ch_ce,), jnp.int32),
                       pltpu.VMEM((L,), jnp.int32),
                       pltpu.VMEM((L,), jnp.int32),
                       pltpu.VMEM((16384,), jnp.int32)],
    )

    phase_d = pl.pallas_call(
        _d_body,
        out_shape=[jax.ShapeDtypeStruct((B, NSEG, L), jnp.int32)],
    )

    phase_e = pl.kernel(
        functools.partial(_e_body, B, N, seg_len, ch_ce, tiles_per_b),
        out_type=[jax.ShapeDtypeStruct((B * tiles_per_b * 65536,), jnp.int32)],
        mesh=mesh,
        compiler_params=pltpu.CompilerParams(needs_layout_passes=False),
        scratch_types=[pltpu.VMEM((ch_ce,), jnp.int32),
                       pltpu.VMEM((L,), jnp.int32),
                       pltpu.VMEM((L,), jnp.int32),
                       pltpu.VMEM((L,), jnp.int32),
                       pltpu.VMEM((65536,), jnp.int32)],
    )

    phase_f = pl.pallas_call(
        _f_body,
        out_shape=jax.ShapeDtypeStruct((B, 1, H, W), jnp.float32),
    )

    return phase_a, phase_b, phase_c, phase_d, phase_e, phase_f


def kernel(events):
    B, N, _ = events.shape
    tiles_per_b = NWORK // B
    phase_a, phase_b, phase_c, phase_d, phase_e, phase_f = _build(B, N)
    xflat = events[:, :, 0].reshape(B * N)
    yflat = events[:, :, 1].reshape(B * N)
    alongx, alongy, packed = phase_a(xflat, yflat)
    axb, ayb, skip = phase_b(alongx.reshape(B, S, 256), alongy.reshape(B, S, 256))
    (occ,) = phase_c(packed, axb.reshape(-1), ayb.reshape(-1))
    (wb,) = phase_d(occ.reshape(B, NSEG, 128, 128), skip)
    (parts,) = phase_e(packed, axb.reshape(-1), ayb.reshape(-1), wb.reshape(-1))
    return phase_f(parts.reshape(B, tiles_per_b, 256, 256))


# speedup vs baseline: 5.8690x; 1.1438x over previous
import functools

import jax
import jax.numpy as jnp
from jax import lax
from jax.experimental import pallas as pl
from jax.experimental.pallas import tpu as pltpu
from jax.experimental.pallas import tpu_sc as plsc

H = W = 256
S = 48
START = 3
NSEG = 35
NDEC = NSEG - 1
NC = 2
NSUB = 16
NWORK = NC * NSUB
L = 16


def _make_mesh():
    return plsc.VectorSubcoreMesh(core_axis_name="c", subcore_axis_name="s")


def _wid():
    return lax.axis_index("s") * NC + lax.axis_index("c")


def _a_body(B, N, seg_len, ch, xp, yp, alongx, alongy, packed, xbuf, ybuf,
            pbuf, hx, hy):
    wid = _wid()
    units = (B * S) // NWORK
    lane = lax.iota(jnp.int32, L)
    lane256 = lane * 256
    ones = jnp.ones((L,), jnp.int32)
    zeros16 = jnp.zeros((L,), jnp.int32)
    n_chunks = seg_len // ch

    for uu in range(units):
        u = wid * units + uu
        base = u * seg_len

        def zb(i, _):
            hx[pl.ds(i * L, L)] = zeros16
            hy[pl.ds(i * L, L)] = zeros16
            return _
        lax.fori_loop(0, 256, zb, None)

        def chunk_body(c, _):
            off = base + c * ch
            pltpu.sync_copy(xp.at[pl.ds(off, ch)], xbuf)
            pltpu.sync_copy(yp.at[pl.ds(off, ch)], ybuf)

            def inner(i, _i):
                xi = xbuf[pl.ds(i * L, L)].astype(jnp.int32)
                yi = ybuf[pl.ds(i * L, L)].astype(jnp.int32)
                plsc.addupdate_scatter(hx, [lane256 + xi], ones)
                plsc.addupdate_scatter(hy, [lane256 + yi], ones)
                pbuf[pl.ds(i * L, L)] = xi + yi * 256
                return _i
            lax.fori_loop(0, ch // L, inner, None, unroll=8)
            pltpu.sync_copy(pbuf, packed.at[pl.ds(off, ch)])
            return _
        lax.fori_loop(0, n_chunks, chunk_body, None)

        def red(g, _):
            accx = hx[pl.ds(g * L, L)]
            accy = hy[pl.ds(g * L, L)]
            for l in range(1, L):
                accx = accx + hx[pl.ds(l * 256 + g * L, L)]
                accy = accy + hy[pl.ds(l * 256 + g * L, L)]
            hx[pl.ds(g * L, L)] = accx
            hy[pl.ds(g * L, L)] = accy
            return _
        lax.fori_loop(0, 16, red, None)
        pltpu.sync_copy(hx.at[pl.ds(0, 256)], alongx.at[pl.ds(u * 256, 256)])
        pltpu.sync_copy(hy.at[pl.ds(0, 256)], alongy.at[pl.ds(u * 256, 256)])


def _median10(v):
    le = jnp.sum((v[..., None, :] <= v[..., :, None]).astype(jnp.int32), axis=-1)
    big = jnp.full_like(v, 3.4e38)
    os5 = jnp.min(jnp.where(le >= 5, v, big), axis=-1)
    os6 = jnp.min(jnp.where(le >= 6, v, big), axis=-1)
    return (os5 + os6) / 2


def _outlier_first(v, thresh):
    med = _median10(v)
    diff = jnp.abs(v - med[..., None])
    mad = _median10(diff)
    mad = jnp.where(mad == 0, jnp.float32(1e-12), mad)
    mz = jnp.float32(0.6745) * diff[..., 0] / mad
    return mz > thresh


def _b_body(seg_len, ax_ref, ay_ref, axb_ref, ayb_ref, skip_ref):
    B = ax_ref.shape[0]
    D = 256

    def stats(cnt):
        a = cnt.astype(jnp.float32)
        n = S * D
        mean = jnp.sum(a, axis=(1, 2)) / jnp.float32(n)
        var = jnp.sum((a - mean[:, None, None]) ** 2, axis=(1, 2)) / jnp.float32(n - 1)
        cvv = mean + 3.0 * jnp.sqrt(var)
        a = jnp.clip(a, 0.0, cvv[:, None, None])
        zc = jnp.zeros((B, S, 2), jnp.float32)
        a2 = jnp.concatenate([zc, a, zc], axis=2)
        zr = jnp.zeros((B, 2, D + 4), jnp.float32)
        a3 = jnp.concatenate([zr, a2, zr], axis=1)
        acc = jnp.zeros_like(a)
        for di in range(5):
            for dj in range(5):
                acc = acc + a3[:, di:di + S, dj:dj + D]
        blur = acc * jnp.float32(0.04)
        w = lax.broadcasted_iota(jnp.int32, (1, 1, D), 2).astype(jnp.float32)
        mean_a = jnp.sum(blur * w, axis=-1) / jnp.float32(seg_len)
        start = mean_a[:, START:START + 1]
        aligned = jnp.round((mean_a - start) - (jnp.float32(D // 2) - start))
        return mean_a, aligned.astype(jnp.int32)

    mx, alx = stats(ax_ref[...])
    my, aly = stats(ay_ref[...])
    axb_ref[...] = jnp.broadcast_to(alx[:, :, None], (B, S, L))
    ayb_ref[...] = jnp.broadcast_to(aly[:, :, None], (B, S, L))
    wx = jnp.concatenate([mx[:, None, si:si + 10] for si in range(4, 4 + NDEC)], axis=1)
    wy = jnp.concatenate([my[:, None, si:si + 10] for si in range(4, 4 + NDEC)], axis=1)
    skip = jnp.logical_or(_outlier_first(wx, 2.0), _outlier_first(wy, 2.0))
    skip_ref[...] = skip.astype(jnp.int32)


def _c_body(B, N, seg_len, ch, packed, axb, ayb, occ, pbuf, abx, aby, occ_s):
    wid = _wid()
    ones = jnp.ones((L,), jnp.int32)
    zeros16 = jnp.zeros((L,), jnp.int32)
    n_units = B * NSEG
    n_rounds = (n_units + NWORK - 1) // NWORK
    n_chunks = seg_len // ch

    for rnd in range(n_rounds):
        u = wid + rnd * NWORK

        @pl.when(u < n_units)
        def _():
            b = u // NSEG
            kk = u % NSEG
            si = kk + START
            pltpu.sync_copy(axb.at[pl.ds((b * S + si) * L, L)], abx)
            pltpu.sync_copy(ayb.at[pl.ds((b * S + si) * L, L)], aby)
            axv = abx[...]
            ayv = aby[...]

            def zb(i, _):
                occ_s[pl.ds(i * L, L)] = zeros16
                return _
            lax.fori_loop(0, 16384 // L, zb, None)

            def chunk_body(c, _):
                pltpu.sync_copy(
                    packed.at[pl.ds(b * N + si * seg_len + c * ch, ch)], pbuf)

                def inner(i, _i):
                    p = pbuf[pl.ds(i * L, L)]
                    xi = lax.bitwise_and(p, 255)
                    yi = lax.shift_right_logical(p, 8)
                    xs = jnp.minimum(jnp.maximum(xi - axv, 0), 255)
                    ys = jnp.minimum(jnp.maximum(yi - ayv, 0), 255)
                    cv = (lax.shift_right_logical(ys, 1) * 128
                          + lax.shift_right_logical(xs, 1))
                    plsc.store_scatter(occ_s, [cv], ones)
                    return _i
                lax.fori_loop(0, ch // L, inner, None, unroll=8)
                return _
            lax.fori_loop(0, n_chunks, chunk_body, None)
            pltpu.sync_copy(occ_s, occ.at[pl.ds(u * 16384, 16384)])


def _d_body(occ_ref, skip_ref, wb_ref):
    B = occ_ref.shape[0]
    skip = skip_ref[...] != 0
    ver = occ_ref[:, 0, :, :]
    s_old = jnp.sum(ver, axis=(1, 2)).astype(jnp.float32)
    active = jnp.ones((B,), jnp.bool_)
    ws = [jnp.ones((B,), jnp.int32)]
    for k in range(1, NSEG):
        o = occ_ref[:, k, :, :]
        cand = jnp.maximum(ver, o)
        s = jnp.sum(cand, axis=(1, 2)).astype(jnp.float32)
        ni = s - s_old
        keep = (ni / s) >= jnp.float32(0.1)
        sk = skip[:, k - 1]
        ap = jnp.logical_and(active, jnp.logical_and(jnp.logical_not(sk), keep))
        ver = jnp.where(ap[:, None, None], cand, ver)
        s_old = jnp.where(ap, s, s_old)
        active = jnp.logical_and(active, jnp.logical_or(sk, keep))
        ws.append(ap.astype(jnp.int32))
    wmat = jnp.stack(ws, axis=1)
    wb_ref[...] = jnp.broadcast_to(wmat[:, :, None], (B, NSEG, L))


def _e_body(B, N, seg_len, ch, tiles_per_b, packed, axb, ayb, wb, parts,
            pbuf, abx, aby, wbuf, cont_s):
    segs_per_tile = -(-NSEG // tiles_per_b)
    wid = _wid()
    ones = jnp.ones((L,), jnp.int32)
    zeros16 = jnp.zeros((L,), jnp.int32)
    n_chunks = seg_len // ch

    @pl.when(wid < B * tiles_per_b)
    def _():
        b = wid // tiles_per_b
        r = wid % tiles_per_b

        def zb(i, _):
            cont_s[pl.ds(i * L, L)] = zeros16
            return _
        lax.fori_loop(0, 65536 // L, zb, None)

        for j in range(segs_per_tile):
            kk_s = None

            kk = r * segs_per_tile + j
            si = kk + START

            @pl.when(kk < NSEG)
            def _():
                pltpu.sync_copy(wb.at[pl.ds((b * NSEG + kk) * L, L)], wbuf)
                wsum = jnp.sum(wbuf[...])

                @pl.when(wsum > 0)
                def _():
                    pltpu.sync_copy(axb.at[pl.ds((b * S + si) * L, L)], abx)
                    pltpu.sync_copy(ayb.at[pl.ds((b * S + si) * L, L)], aby)
                    axv = abx[...]
                    ayv = aby[...]

                    def chunk_body(c, _):
                        pltpu.sync_copy(
                            packed.at[pl.ds(b * N + si * seg_len + c * ch, ch)],
                            pbuf)

                        def inner(i, _i):
                            p = pbuf[pl.ds(i * L, L)]
                            xi = lax.bitwise_and(p, 255)
                            yi = lax.shift_right_logical(p, 8)
                            xs = jnp.minimum(jnp.maximum(xi - axv, 0), 255)
                            ys = jnp.minimum(jnp.maximum(yi - ayv, 0), 255)
                            plsc.addupdate_scatter(cont_s, [ys * 256 + xs], ones)
                            return _i
                        lax.fori_loop(0, ch // L, inner, None, unroll=8)
                        return _
                    lax.fori_loop(0, n_chunks, chunk_body, None)
        pltpu.sync_copy(cont_s, parts.at[pl.ds(wid * 65536, 65536)])


def _f_body(parts_ref, out_ref):
    B = parts_ref.shape[0]
    cont = jnp.sum(parts_ref[...], axis=1).astype(jnp.float32)
    n = H * W
    mean = jnp.sum(cont, axis=(1, 2)) / jnp.float32(n)
    var = jnp.sum((cont - mean[:, None, None]) ** 2, axis=(1, 2)) / jnp.float32(n - 1)
    cv = mean + 3.0 * jnp.sqrt(var)
    c = jnp.clip(cont, 0.0, cv[:, None, None]) / cv[:, None, None]
    out_ref[...] = c[:, None, :, :]


@functools.lru_cache(maxsize=4)
def _build(B, N):
    seg_len = N // S
    ch_a = 4000 if seg_len % 4000 == 0 else seg_len
    ch_ce = 8000 if seg_len % 8000 == 0 else seg_len
    tiles_per_b = NWORK // B
    mesh = _make_mesh()

    phase_a = pl.kernel(
        functools.partial(_a_body, B, N, seg_len, ch_a),
        out_type=[jax.ShapeDtypeStruct((B * S * 256,), jnp.int32),
                  jax.ShapeDtypeStruct((B * S * 256,), jnp.int32),
                  jax.ShapeDtypeStruct((B * N,), jnp.int32)],
        mesh=mesh,
        compiler_params=pltpu.CompilerParams(needs_layout_passes=False),
        scratch_types=[pltpu.VMEM((ch_a,), jnp.float32),
                       pltpu.VMEM((ch_a,), jnp.float32),
                       pltpu.VMEM((ch_a,), jnp.int32),
                       pltpu.VMEM((16 * 256,), jnp.int32),
                       pltpu.VMEM((16 * 256,), jnp.int32)],
    )

    phase_b = pl.pallas_call(
        functools.partial(_b_body, seg_len),
        out_shape=[jax.ShapeDtypeStruct((B, S, L), jnp.int32),
                   jax.ShapeDtypeStruct((B, S, L), jnp.int32),
                   jax.ShapeDtypeStruct((B, NDEC), jnp.int32)],
    )

    phase_c = pl.kernel(
        functools.partial(_c_body, B, N, seg_len, ch_ce),
        out_type=[jax.ShapeDtypeStruct((B * NSEG * 16384,), jnp.int32)],
        mesh=mesh,
        compiler_params=pltpu.CompilerParams(needs_layout_passes=False),
        scratch_types=[pltpu.VMEM((ch_ce,), jnp.int32),
                       pltpu.VMEM((L,), jnp.int32),
                       pltpu.VMEM((L,), jnp.int32),
                       pltpu.VMEM((16384,), jnp.int32)],
    )

    phase_d = pl.pallas_call(
        _d_body,
        out_shape=[jax.ShapeDtypeStruct((B, NSEG, L), jnp.int32)],
    )

    phase_e = pl.kernel(
        functools.partial(_e_body, B, N, seg_len, ch_ce, tiles_per_b),
        out_type=[jax.ShapeDtypeStruct((B * tiles_per_b * 65536,), jnp.int32)],
        mesh=mesh,
        compiler_params=pltpu.CompilerParams(needs_layout_passes=False),
        scratch_types=[pltpu.VMEM((ch_ce,), jnp.int32),
                       pltpu.VMEM((L,), jnp.int32),
                       pltpu.VMEM((L,), jnp.int32),
                       pltpu.VMEM((L,), jnp.int32),
                       pltpu.VMEM((65536,), jnp.int32)],
    )

    phase_f = pl.pallas_call(
        _f_body,
        out_shape=jax.ShapeDtypeStruct((B, 1, H, W), jnp.float32),
    )

    return phase_a, phase_b, phase_c, phase_d, phase_e, phase_f


def kernel(events):
    B, N, _ = events.shape
    tiles_per_b = NWORK // B
    phase_a, phase_b, phase_c, phase_d, phase_e, phase_f = _build(B, N)
    xflat = events[:, :, 0].reshape(B * N)
    yflat = events[:, :, 1].reshape(B * N)
    alongx, alongy, packed = phase_a(xflat, yflat)
    axb, ayb, skip = phase_b(alongx.reshape(B, S, 256), alongy.reshape(B, S, 256))
    (occ,) = phase_c(packed, axb.reshape(-1), ayb.reshape(-1))
    (wb,) = phase_d(occ.reshape(B, NSEG, 128, 128), skip)
    (parts,) = phase_e(packed, axb.reshape(-1), ayb.reshape(-1), wb.reshape(-1))
    return phase_f(parts.reshape(B, tiles_per_b, 256, 256))
